# baseline (speedup 1.0000x reference)
import jax
import jax.numpy as jnp
from jax import lax
from jax.experimental import pallas as pl
from jax.experimental.pallas import tpu as pltpu
from jax.experimental.pallas import tpu_sc as plsc

D = 64
B = 16384

NC = 2
NS = 16
L = 16
NW = NC * NS
B_PER_W = B // NW
CHUNK = 256
N_CHUNK = B_PER_W // CHUNK
ROWS_PER_ITER = 16


def _body(hd, hg, di, dg, out, di_v, dg_v, dis_v, drg_v, out_v, tr_v, sem1, sem2):
    wid = lax.axis_index("s") * NC + lax.axis_index("c")
    base = wid * B_PER_W

    with jax.named_scope("p0_idx_load"):
        pltpu.sync_copy(di.at[pl.ds(base, B_PER_W)], di_v)
        pltpu.sync_copy(dg.at[pl.ds(base, B_PER_W)], dg_v)

    lane = lax.iota(jnp.int32, L)

    for ch in range(N_CHUNK):
        coff = ch * CHUNK

        def issue_body(it, carry):
            ibase = coff + it * ROWS_PER_ITER
            iv = di_v[pl.ds(ibase, ROWS_PER_ITER)]
            gv = dg_v[pl.ds(ibase, ROWS_PER_ITER)]
            for j in range(ROWS_PER_ITER):
                i = ibase + j
                pltpu.async_copy(
                    hd.at[pl.ds(iv[j], 1)], dis_v.at[pl.ds(i - coff, 1)], sem1
                )
                pltpu.async_copy(
                    hg.at[pl.ds(gv[j], 1)], drg_v.at[pl.ds(i - coff, 1)], sem2
                )
            return carry

        with jax.named_scope("p1_issue"):
            lax.fori_loop(0, CHUNK // ROWS_PER_ITER, issue_body, 0)

        with jax.named_scope("p2_drain"):
            pltpu.make_async_copy(hd.at[pl.ds(0, CHUNK)], dis_v, sem1).wait()
            pltpu.make_async_copy(hg.at[pl.ds(0, CHUNK)], drg_v, sem2).wait()

        def blk_body(blk, carry):
            rbase = blk * L
            for r in range(L):
                row = rbase + r
                acc = dis_v[row, pl.ds(0, L)] * drg_v[row, pl.ds(0, L)]
                for c in range(1, D // L):
                    acc = acc + dis_v[row, pl.ds(c * L, L)] * drg_v[row, pl.ds(c * L, L)]
                plsc.store_scatter(tr_v, [lane * L + r], acc)
            tot = tr_v[pl.ds(0, L)]
            for i in range(1, L):
                tot = tot + tr_v[pl.ds(i * L, L)]
            out_v[pl.ds(coff + rbase, L)] = tot
            return carry

        with jax.named_scope("p3_compute"):
            lax.fori_loop(0, CHUNK // L, blk_body, 0)

    with jax.named_scope("p4_writeback"):
        pltpu.sync_copy(out_v, out.at[pl.ds(base, B_PER_W)])


@jax.jit
def _run(h_disease, h_drug, disease_indices, drug_indices):
    k = pl.kernel(
        _body,
        out_type=jax.ShapeDtypeStruct((B,), jnp.float32),
        mesh=plsc.VectorSubcoreMesh(core_axis_name="c", subcore_axis_name="s"),
        compiler_params=pltpu.CompilerParams(
            needs_layout_passes=False, use_tc_tiling_on_sc=True
        ),
        scratch_types=[
            pltpu.VMEM((B_PER_W,), jnp.int32),
            pltpu.VMEM((B_PER_W,), jnp.int32),
            pltpu.VMEM((CHUNK, D), jnp.float32),
            pltpu.VMEM((CHUNK, D), jnp.float32),
            pltpu.VMEM((B_PER_W,), jnp.float32),
            pltpu.VMEM((L * L,), jnp.float32),
            pltpu.SemaphoreType.DMA,
            pltpu.SemaphoreType.DMA,
        ],
    )
    return k(h_disease, h_drug, disease_indices, drug_indices)


def kernel(h_disease, h_drug, disease_indices, drug_indices):
    return _run(
        h_disease,
        h_drug,
        jnp.asarray(disease_indices, jnp.int32),
        jnp.asarray(drug_indices, jnp.int32),
    )


# speedup vs baseline: 1.0244x; 1.0244x over previous
import jax
import jax.numpy as jnp
from jax import lax
from jax.experimental import pallas as pl
from jax.experimental.pallas import tpu as pltpu
from jax.experimental.pallas import tpu_sc as plsc

D = 64
DP = 128
V = 100000
B = 16384

NC = 2
NS = 16
L = 16
NW = NC * NS
B_PER_W = B // NW
CHUNK = 256
N_CHUNK = B_PER_W // CHUNK
ROWS_PER_ITER = 16

TBL = 2048


def _tr_body(hd_ref, hg_ref, od_ref, og_ref):
    od_ref[...] = hd_ref[...].T.reshape(TBL // 8, 8, D)
    og_ref[...] = hg_ref[...].T.reshape(TBL // 8, 8, D)


def _transpose(hdt, hgt):
    grid = (pl.cdiv(V, TBL),)
    return pl.pallas_call(
        _tr_body,
        grid=grid,
        in_specs=[
            pl.BlockSpec((D, TBL), lambda i: (0, i)),
            pl.BlockSpec((D, TBL), lambda i: (0, i)),
        ],
        out_specs=[
            pl.BlockSpec((TBL // 8, 8, D), lambda i: (i, 0, 0)),
            pl.BlockSpec((TBL // 8, 8, D), lambda i: (i, 0, 0)),
        ],
        out_shape=[
            jax.ShapeDtypeStruct((V // 8, 8, D), jnp.float32),
            jax.ShapeDtypeStruct((V // 8, 8, D), jnp.float32),
        ],
    )(hdt, hgt)


def _body(hd, hg, di, dg, out, di_v, dg_v, dis_v, drg_v, out_v, tr_v, sem1, sem2):
    wid = lax.axis_index("s") * NC + lax.axis_index("c")
    base = wid * B_PER_W

    pltpu.sync_copy(di.at[pl.ds(base, B_PER_W)], di_v)
    pltpu.sync_copy(dg.at[pl.ds(base, B_PER_W)], dg_v)

    lane = lax.iota(jnp.int32, L)

    for ch in range(N_CHUNK):
        coff = ch * CHUNK

        def issue_body(it, carry):
            ibase = coff + it * ROWS_PER_ITER
            iv = di_v[pl.ds(ibase, ROWS_PER_ITER)]
            gv = dg_v[pl.ds(ibase, ROWS_PER_ITER)]
            for j in range(ROWS_PER_ITER):
                i = ibase - coff + j
                pltpu.async_copy(
                    hd.at[pl.ds(iv[j] >> 3, 1), pl.ds(iv[j] & 7, 1)],
                    dis_v.at[pl.ds(i >> 3, 1), pl.ds(i & 7, 1)],
                    sem1,
                )
                pltpu.async_copy(
                    hg.at[pl.ds(gv[j] >> 3, 1), pl.ds(gv[j] & 7, 1)],
                    drg_v.at[pl.ds(i >> 3, 1), pl.ds(i & 7, 1)],
                    sem2,
                )
            return carry

        lax.fori_loop(0, CHUNK // ROWS_PER_ITER, issue_body, 0)

        pltpu.make_async_copy(hd.at[pl.ds(0, CHUNK // 8)], dis_v, sem1).wait()
        pltpu.make_async_copy(hg.at[pl.ds(0, CHUNK // 8)], drg_v, sem2).wait()

        def blk_body(blk, carry):
            rbase = blk * L
            for r in range(L):
                row = rbase + r
                q = row >> 3
                t = row & 7
                acc = dis_v[q, t, pl.ds(0, L)] * drg_v[q, t, pl.ds(0, L)]
                for c in range(1, D // L):
                    acc = acc + dis_v[q, t, pl.ds(c * L, L)] * drg_v[q, t, pl.ds(c * L, L)]
                plsc.store_scatter(tr_v, [lane * L + r], acc)
            tot = tr_v[pl.ds(0, L)]
            for i in range(1, L):
                tot = tot + tr_v[pl.ds(i * L, L)]
            out_v[pl.ds(coff + rbase, L)] = tot
            return carry

        lax.fori_loop(0, CHUNK // L, blk_body, 0)

    pltpu.sync_copy(out_v, out.at[pl.ds(base, B_PER_W)])


@jax.jit
def _run(h_disease_t, h_drug_t, disease_indices, drug_indices):
    htd, htg = _transpose(h_disease_t, h_drug_t)
    k = pl.kernel(
        _body,
        out_type=jax.ShapeDtypeStruct((B,), jnp.float32),
        mesh=plsc.VectorSubcoreMesh(core_axis_name="c", subcore_axis_name="s"),
        compiler_params=pltpu.CompilerParams(
            needs_layout_passes=False, use_tc_tiling_on_sc=True
        ),
        scratch_types=[
            pltpu.VMEM((B_PER_W,), jnp.int32),
            pltpu.VMEM((B_PER_W,), jnp.int32),
            pltpu.VMEM((CHUNK // 8, 8, D), jnp.float32),
            pltpu.VMEM((CHUNK // 8, 8, D), jnp.float32),
            pltpu.VMEM((B_PER_W,), jnp.float32),
            pltpu.VMEM((L * L,), jnp.float32),
            pltpu.SemaphoreType.DMA,
            pltpu.SemaphoreType.DMA,
        ],
    )
    return k(htd, htg, disease_indices, drug_indices)


def kernel(h_disease, h_drug, disease_indices, drug_indices):
    return _run(
        h_disease.T,
        h_drug.T,
        jnp.asarray(disease_indices, jnp.int32),
        jnp.asarray(drug_indices, jnp.int32),
    )


# speedup vs baseline: 1.2496x; 1.2198x over previous
import jax
import jax.numpy as jnp
from jax import lax
from jax.experimental import pallas as pl
from jax.experimental.pallas import tpu as pltpu
from jax.experimental.pallas import tpu_sc as plsc

D = 64
DP = 128
V = 100000
B = 16384

NC = 2
NS = 16
L = 16
NW = NC * NS
B_PER_W = B // NW
CHUNK = 256
N_CHUNK = B_PER_W // CHUNK
ROWS_PER_ITER = 16

TBL = 8192


def _tr_body(hd_ref, hg_ref, od_ref, og_ref):
    od_ref[...] = hd_ref[...].T.reshape(TBL // 8, 8, D)
    og_ref[...] = hg_ref[...].T.reshape(TBL // 8, 8, D)


def _transpose(hdt, hgt):
    grid = (pl.cdiv(V, TBL),)
    return pl.pallas_call(
        _tr_body,
        grid=grid,
        in_specs=[
            pl.BlockSpec((D, TBL), lambda i: (0, i)),
            pl.BlockSpec((D, TBL), lambda i: (0, i)),
        ],
        out_specs=[
            pl.BlockSpec((TBL // 8, 8, D), lambda i: (i, 0, 0)),
            pl.BlockSpec((TBL // 8, 8, D), lambda i: (i, 0, 0)),
        ],
        out_shape=[
            jax.ShapeDtypeStruct((V // 8, 8, D), jnp.float32),
            jax.ShapeDtypeStruct((V // 8, 8, D), jnp.float32),
        ],
    )(hdt, hgt)


def _body(hd, hg, di, dg, out, di_v, dg_v, dis_v, drg_v, out_v, tr_v, sem1, sem2):
    wid = lax.axis_index("s") * NC + lax.axis_index("c")
    base = wid * B_PER_W

    pltpu.sync_copy(di.at[pl.ds(base, B_PER_W)], di_v)
    pltpu.sync_copy(dg.at[pl.ds(base, B_PER_W)], dg_v)

    lane = lax.iota(jnp.int32, L)

    for ch in range(N_CHUNK):
        coff = ch * CHUNK

        def issue_body(it, carry):
            ibase = coff + it * ROWS_PER_ITER
            iv = di_v[pl.ds(ibase, ROWS_PER_ITER)]
            gv = dg_v[pl.ds(ibase, ROWS_PER_ITER)]
            for j in range(ROWS_PER_ITER):
                i = ibase - coff + j
                pltpu.async_copy(
                    hd.at[pl.ds(iv[j] >> 3, 1), pl.ds(iv[j] & 7, 1)],
                    dis_v.at[pl.ds(i >> 3, 1), pl.ds(i & 7, 1)],
                    sem1,
                )
                pltpu.async_copy(
                    hg.at[pl.ds(gv[j] >> 3, 1), pl.ds(gv[j] & 7, 1)],
                    drg_v.at[pl.ds(i >> 3, 1), pl.ds(i & 7, 1)],
                    sem2,
                )
            return carry

        lax.fori_loop(0, CHUNK // ROWS_PER_ITER, issue_body, 0)

        pltpu.make_async_copy(hd.at[pl.ds(0, CHUNK // 8)], dis_v, sem1).wait()
        pltpu.make_async_copy(hg.at[pl.ds(0, CHUNK // 8)], drg_v, sem2).wait()

        def blk_body(blk, carry):
            rbase = blk * L
            for r in range(L):
                row = rbase + r
                q = row >> 3
                t = row & 7
                acc = dis_v[q, t, pl.ds(0, L)] * drg_v[q, t, pl.ds(0, L)]
                for c in range(1, D // L):
                    acc = acc + dis_v[q, t, pl.ds(c * L, L)] * drg_v[q, t, pl.ds(c * L, L)]
                plsc.store_scatter(tr_v, [lane * L + r], acc)
            tot = tr_v[pl.ds(0, L)]
            for i in range(1, L):
                tot = tot + tr_v[pl.ds(i * L, L)]
            out_v[pl.ds(coff + rbase, L)] = tot
            return carry

        lax.fori_loop(0, CHUNK // L, blk_body, 0)

    pltpu.sync_copy(out_v, out.at[pl.ds(base, B_PER_W)])


@jax.jit
def _run(h_disease_t, h_drug_t, disease_indices, drug_indices):
    htd, htg = _transpose(h_disease_t, h_drug_t)
    k = pl.kernel(
        _body,
        out_type=jax.ShapeDtypeStruct((B,), jnp.float32),
        mesh=plsc.VectorSubcoreMesh(core_axis_name="c", subcore_axis_name="s"),
        compiler_params=pltpu.CompilerParams(
            needs_layout_passes=False, use_tc_tiling_on_sc=True
        ),
        scratch_types=[
            pltpu.VMEM((B_PER_W,), jnp.int32),
            pltpu.VMEM((B_PER_W,), jnp.int32),
            pltpu.VMEM((CHUNK // 8, 8, D), jnp.float32),
            pltpu.VMEM((CHUNK // 8, 8, D), jnp.float32),
            pltpu.VMEM((B_PER_W,), jnp.float32),
            pltpu.VMEM((L * L,), jnp.float32),
            pltpu.SemaphoreType.DMA,
            pltpu.SemaphoreType.DMA,
        ],
    )
    return k(htd, htg, disease_indices, drug_indices)


def kernel(h_disease, h_drug, disease_indices, drug_indices):
    return _run(
        h_disease.T,
        h_drug.T,
        jnp.asarray(disease_indices, jnp.int32),
        jnp.asarray(drug_indices, jnp.int32),
    )
